# baseline (speedup 1.0000x reference)
import jax
import jax.numpy as jnp
from jax import lax
from jax.experimental import pallas as pl
from jax.experimental.pallas import tpu as pltpu
from jax.experimental.pallas import tpu_sc as plsc

VOCAB = 1_000_000
SEQ = 200
D = 64
BATCH = 4096

NC, NS = 2, 16
NW = NC * NS
L = 16
BLK = 128
BPS = BATCH // BLK
N_BLOCKS = SEQ * BPS
BLK_PER_W = N_BLOCKS // NW


def _emb_kernel(idx_hbm, table_hbm, pos_hbm, out_hbm,
                idx_v, pair_v, out_v, pos_v, psplat_v,
                in_s0, in_s1, g_s0, g_s1, o_s0, o_s1):
    in_sem = (in_s0, in_s1)
    g_sem = (g_s0, g_s1)
    out_sem = (o_s0, o_s1)
    wid = lax.axis_index("s") * NC + lax.axis_index("c")
    base = wid * BLK_PER_W

    pltpu.sync_copy(pos_hbm, pos_v)

    def idx_desc(g, s):
        blk = base + g
        return pltpu.make_async_copy(
            idx_hbm.at[blk // BPS, pl.ds((blk % BPS) * BLK, BLK)],
            idx_v.at[s], in_sem[s])

    def gather_desc(s):
        return pltpu.make_async_copy(
            table_hbm.at[idx_v.at[s]], pair_v.at[s], g_sem[s])

    def out_desc(g, s):
        blk = base + g
        return pltpu.make_async_copy(
            out_v.at[s],
            out_hbm.at[blk // BPS, :, pl.ds((blk % BPS) * BLK, BLK)],
            out_sem[s])

    def build_pos_splats(g):
        srow = (base + g) // BPS
        for k in range(D // L):
            prow = pos_v[srow, pl.ds(k * L, L)]
            for lane in range(L):
                d = k * L + lane
                psplat_v[d, :] = jnp.broadcast_to(prow[lane], (L,))

    def transpose_add(s):
        for d in range(D):
            pv = psplat_v[d, :]
            col = jnp.full((L,), d, jnp.int32)
            for grp in range(BLK // L):
                rows = jax.lax.iota(jnp.int32, L) + (grp * L)
                vals = plsc.load_gather(pair_v.at[s], [rows, col])
                out_v[s, d, pl.ds(grp * L, L)] = vals + pv

    def process(g, s):
        idx_desc(g, s).wait()
        gather_desc(s).start()
        o = 1 - s

        @pl.when(g + 1 < BLK_PER_W)
        def _():
            idx_desc(g + 1, o).start()

        build_pos_splats(g)

        @pl.when(g > 0)
        def _():
            out_desc(g - 1, o).wait()

        gather_desc(s).wait()
        transpose_add(s)
        out_desc(g, s).start()

    idx_desc(0, 0).start()

    def body(i, _):
        process(2 * i, 0)
        process(2 * i + 1, 1)
        return _

    lax.fori_loop(0, BLK_PER_W // 2, body, None)
    out_desc(BLK_PER_W - 1, 1).wait()


@jax.jit
def _embed(idx_t, table_padded, pos_table):
    mesh = plsc.VectorSubcoreMesh(
        core_axis_name="c", subcore_axis_name="s", num_cores=NC, num_subcores=NS
    )
    fn = pl.kernel(
        _emb_kernel,
        out_type=jax.ShapeDtypeStruct((SEQ, D, BATCH), jnp.float32),
        mesh=mesh,
        scratch_types=[
            pltpu.VMEM((2, BLK), jnp.int32),
            pltpu.VMEM((2, BLK, 2 * D), jnp.float32),
            pltpu.VMEM((2, D, BLK), jnp.float32),
            pltpu.VMEM((SEQ, D), jnp.float32),
            pltpu.VMEM((D, L), jnp.float32),
            pltpu.SemaphoreType.DMA,
            pltpu.SemaphoreType.DMA,
            pltpu.SemaphoreType.DMA,
            pltpu.SemaphoreType.DMA,
            pltpu.SemaphoreType.DMA,
            pltpu.SemaphoreType.DMA,
        ],
        compiler_params=pltpu.CompilerParams(needs_layout_passes=False),
    )
    return fn(idx_t, table_padded, pos_table)


def kernel(inputs, token_table, pos_table):
    idx_t = inputs.astype(jnp.int32).T
    table_padded = jnp.pad(token_table.astype(jnp.float32), ((0, 0), (0, D)))
    out = _embed(idx_t, table_padded, pos_table.astype(jnp.float32))
    return out.transpose(2, 0, 1)


# speedup vs baseline: 1.8997x; 1.8997x over previous
import jax
import jax.numpy as jnp
from jax import lax
from jax.experimental import pallas as pl
from jax.experimental.pallas import tpu as pltpu
from jax.experimental.pallas import tpu_sc as plsc

VOCAB = 1_000_000
SEQ = 200
D = 64
BATCH = 4096

NC, NS = 2, 16
NW = NC * NS
L = 16
B_TOTAL = BATCH * SEQ
B_PER_W = B_TOTAL // NW
IBLK = 128
CHUNK = 256
KBLK = CHUNK // IBLK
N_CHUNKS = B_PER_W // CHUNK
POS_TILE = 6400
NFILL = POS_TILE // CHUNK


def _emb_kernel(idx_hbm, table_hbm, pos_hbm, out_hbm, idx_v, buf_v, out_v,
                in_s0, in_s1, g_s0, g_s1, o_s0, o_s1):
    in_sem = (in_s0, in_s1)
    g_sem = (g_s0, g_s1)
    out_sem = (o_s0, o_s1)
    wid = lax.axis_index("s") * NC + lax.axis_index("c")
    base = wid * B_PER_W

    def in_descs(g, s):
        row0 = base + g * CHUNK
        pos0 = lax.rem(g, NFILL) * CHUNK
        return (
            pltpu.make_async_copy(
                idx_hbm.at[pl.ds(row0, CHUNK)], idx_v.at[s], in_sem[s]),
            pltpu.make_async_copy(
                pos_hbm.at[pl.ds(pos0, CHUNK)], out_v.at[s], in_sem[s]),
        )

    def gather_descs(s):
        return [
            pltpu.make_async_copy(
                table_hbm.at[idx_v.at[s, pl.ds(j * IBLK, IBLK)]],
                buf_v.at[pl.ds(j * IBLK, IBLK)],
                g_sem[s])
            for j in range(KBLK)
        ]

    def out_desc(g, s):
        row0 = base + g * CHUNK
        return pltpu.make_async_copy(
            out_v.at[s], out_hbm.at[pl.ds(row0, CHUNK)], out_sem[s])

    def add_low_halves(s):
        def rbody(q, _):
            for u in range(4):
                r = q * 4 + u
                for k in range(D // L):
                    plsc.addupdate(
                        out_v.at[s, r, pl.ds(k * L, L)],
                        buf_v[r, pl.ds(k * L, L)],
                    )
            return _
        lax.fori_loop(0, CHUNK // 4, rbody, None)

    def process(g, s):
        for d in in_descs(g, s):
            d.wait()
        gd = gather_descs(s)
        for d in gd:
            d.start()
        o = 1 - s

        @pl.when(g > 0)
        def _():
            out_desc(g - 1, o).wait()

        @pl.when(g + 1 < N_CHUNKS)
        def _():
            for d in in_descs(g + 1, o):
                d.start()
        for d in gd:
            d.wait()
        add_low_halves(s)
        out_desc(g, s).start()

    for d in in_descs(0, 0):
        d.start()

    def body(i, _):
        process(2 * i, 0)
        process(2 * i + 1, 1)
        return _

    lax.fori_loop(0, N_CHUNKS // 2, body, None)
    out_desc(N_CHUNKS - 1, 1).wait()


@jax.jit
def _embed(idx_flat, table_padded, pos_tiled):
    mesh = plsc.VectorSubcoreMesh(
        core_axis_name="c", subcore_axis_name="s", num_cores=NC, num_subcores=NS
    )
    fn = pl.kernel(
        _emb_kernel,
        out_type=jax.ShapeDtypeStruct((B_TOTAL, D), jnp.float32),
        mesh=mesh,
        scratch_types=[
            pltpu.VMEM((2, CHUNK), jnp.int32),
            pltpu.VMEM((CHUNK, 2 * D), jnp.float32),
            pltpu.VMEM((2, CHUNK, D), jnp.float32),
            pltpu.SemaphoreType.DMA,
            pltpu.SemaphoreType.DMA,
            pltpu.SemaphoreType.DMA,
            pltpu.SemaphoreType.DMA,
            pltpu.SemaphoreType.DMA,
            pltpu.SemaphoreType.DMA,
        ],
    )
    return fn(idx_flat, table_padded, pos_tiled)


def kernel(inputs, token_table, pos_table):
    idx_flat = inputs.astype(jnp.int32).reshape(B_TOTAL)
    table_padded = jnp.pad(token_table.astype(jnp.float32), ((0, 0), (0, D)))
    pos_tiled = jnp.tile(pos_table.astype(jnp.float32), (POS_TILE // SEQ, 1))
    out = _embed(idx_flat, table_padded, pos_tiled)
    return out.reshape(BATCH, SEQ, D)
